# baseline (speedup 1.0000x reference)
import functools

import jax
import jax.numpy as jnp
from jax import lax
from jax.experimental import pallas as pl
from jax.experimental.pallas import tpu as pltpu
from jax.experimental.pallas import tpu_sc as plsc

_BETA = 0.25

_B, _T, _C = 8, 1024, 256
_K = 8192
_M = _B * _T

_TM = 512
_TN = 8192
_NI = _M // _TM
_NJ = _K // _TN

_NC, _NS = 2, 16
_NW = _NC * _NS
_ROWS_PER_W = _M // _NW


def _vq_body(z_ref, cb_ref, sim_ref, ids_ref, dn_ref,
             best_ref, bidx_ref, cbn_ref, ncb_ref, zn_ref, nz_ref):
    i = pl.program_id(0)
    j = pl.program_id(1)

    @pl.when(jnp.logical_and(i == 0, j == 0))
    def _():
        cb_all = cb_ref[...]
        ncb_col = jnp.sum(cb_all * cb_all, axis=1, keepdims=True)
        cbn_ref[...] = cb_all * (1.0 / jnp.sqrt(ncb_col))
        ncb_ref[...] = ncb_col.reshape(1, _K)

    @pl.when(j == 0)
    def _():
        z_all = z_ref[...]
        nz_col = jnp.sum(z_all * z_all, axis=1, keepdims=True)
        nz_ref[...] = nz_col
        zn_ref[...] = z_all * (1.0 / jnp.sqrt(nz_col))

    z = z_ref[...]
    cb = cb_ref[pl.ds(j * _TN, _TN), :]
    nz = nz_ref[...]
    ncb = ncb_ref[:, pl.ds(j * _TN, _TN)]

    zn = zn_ref[...]
    cbn = cbn_ref[pl.ds(j * _TN, _TN), :]
    sim_ref[...] = lax.dot_general(zn, cbn, (((1,), (1,)), ((), ())),
                                   preferred_element_type=jnp.float32)

    dot = lax.dot_general(z, cb, (((1,), (1,)), ((), ())),
                          preferred_element_type=jnp.float32)

    m = dot[:, 0:1]
    cand = jnp.zeros((_TM, 1), jnp.int32)

    @pl.when(j == 0)
    def _():
        best_ref[...] = m
        bidx_ref[...] = cand

    @pl.when(j > 0)
    def _():
        better = m < best_ref[...]
        bidx_ref[...] = jnp.where(better, cand, bidx_ref[...])
        best_ref[...] = jnp.minimum(m, best_ref[...])

    @pl.when(j == _NJ - 1)
    def _():
        ids_ref[...] = bidx_ref[...].reshape(1, _TM, 1)
        dn_ref[...] = jnp.sqrt(jnp.maximum(best_ref[...], 0.0)).reshape(1, _TM, 1)


def _vq_distance_stage(z2d, codebook):
    return pl.pallas_call(
        _vq_body,
        grid=(_NI, _NJ),
        in_specs=[
            pl.BlockSpec((_TM, _C), lambda i, j: (i, 0)),
            pl.BlockSpec((_K, _C), lambda i, j: (0, 0)),
        ],
        out_specs=[
            pl.BlockSpec((_TM, _TN), lambda i, j: (i, j)),
            pl.BlockSpec((1, _TM, 1), lambda i, j: (i, 0, 0)),
            pl.BlockSpec((1, _TM, 1), lambda i, j: (i, 0, 0)),
        ],
        out_shape=[
            jax.ShapeDtypeStruct((_M, _K), jnp.float32),
            jax.ShapeDtypeStruct((_NI, _TM, 1), jnp.int32),
            jax.ShapeDtypeStruct((_NI, _TM, 1), jnp.float32),
        ],
        scratch_shapes=[
            pltpu.VMEM((_TM, 1), jnp.float32),
            pltpu.VMEM((_TM, 1), jnp.int32),
            pltpu.VMEM((_K, _C), jnp.float32),
            pltpu.VMEM((1, _K), jnp.float32),
            pltpu.VMEM((_TM, _C), jnp.float32),
            pltpu.VMEM((_TM, 1), jnp.float32),
        ],
    )(z2d, codebook)


@functools.lru_cache(maxsize=1)
def _build_gather_rows():
    @functools.partial(
        pl.kernel,
        mesh=plsc.VectorSubcoreMesh(core_axis_name="c", subcore_axis_name="s"),
        out_type=jax.ShapeDtypeStruct((_M, _C), jnp.float32),
        scratch_types=[
            pltpu.VMEM((_ROWS_PER_W,), jnp.int32),
            pltpu.VMEM((_ROWS_PER_W, _C), jnp.float32),
            pltpu.SemaphoreType.DMA,
        ],
    )
    def _gather_rows(table_hbm, idx_hbm, out_hbm, idx_v, rows_v, sem):
        wid = lax.axis_index("s") * _NC + lax.axis_index("c")
        base = wid * _ROWS_PER_W
        pltpu.sync_copy(idx_hbm.at[pl.ds(base, _ROWS_PER_W)], idx_v)
        pltpu.async_copy(table_hbm.at[idx_v], rows_v, sem).wait()
        pltpu.sync_copy(rows_v, out_hbm.at[pl.ds(base, _ROWS_PER_W)])

    return _gather_rows


def kernel(z_e, codebook):
    z2d = z_e.reshape(_M, _C)
    sim, ids3, dn3 = _vq_distance_stage(z2d, codebook)
    ids = ids3.reshape(_M)
    z_q = _build_gather_rows()(codebook, ids)
    norms = dn3.reshape(_M)
    loss_vq = (1.0 + _BETA) * jnp.mean(norms)
    return (
        z_q.reshape(_B, _T, _C),
        sim.reshape(_B, _T, _K),
        ids.reshape(_B, _T),
        loss_vq,
    )


# speedup vs baseline: 2.8620x; 2.8620x over previous
import functools

import jax
import jax.numpy as jnp
from jax import lax
from jax.experimental import pallas as pl
from jax.experimental.pallas import tpu as pltpu
from jax.experimental.pallas import tpu_sc as plsc

_BETA = 0.25

_B, _T, _C = 8, 1024, 256
_K = 8192
_M = _B * _T

_TM = 512
_TN = 8192
_NI = _M // _TM
_NJ = _K // _TN

_NC, _NS = 2, 16
_NW = _NC * _NS
_ROWS_PER_W = _M // _NW


def _vq_body(z_ref, cb_ref, sim_ref, ids_ref, dn_ref,
             best_ref, bidx_ref, ncb_ref, rcb_ref, nz_ref, rz_ref):
    i = pl.program_id(0)
    j = pl.program_id(1)

    @pl.when(jnp.logical_and(i == 0, j == 0))
    def _():
        cb_all = cb_ref[...]
        ncb_row = jnp.sum(cb_all * cb_all, axis=1, keepdims=True).reshape(1, _K)
        ncb_ref[...] = ncb_row
        rcb_ref[...] = 1.0 / jnp.sqrt(ncb_row)

    @pl.when(j == 0)
    def _():
        z_all = z_ref[...]
        nz_col = jnp.sum(z_all * z_all, axis=1, keepdims=True)
        nz_ref[...] = nz_col
        rz_ref[...] = 1.0 / jnp.sqrt(nz_col)

    z = z_ref[...]
    cb = cb_ref[pl.ds(j * _TN, _TN), :]
    nz = nz_ref[...]
    ncb = ncb_ref[:, pl.ds(j * _TN, _TN)]

    dot = lax.dot_general(z, cb, (((1,), (1,)), ((), ())),
                          preferred_element_type=jnp.float32)

    sim_ref[...] = (dot * rz_ref[...]) * rcb_ref[:, pl.ds(j * _TN, _TN)]

    dist = (-2.0 * dot + nz) + ncb

    run_min = dist[:, 0:128]
    run_col = lax.broadcasted_iota(jnp.int32, (_TM, 128), 1) + j * _TN
    for c in range(1, _TN // 128):
        chunk = dist[:, c * 128:(c + 1) * 128]
        ccol = lax.broadcasted_iota(jnp.int32, (_TM, 128), 1) + (j * _TN + c * 128)
        lt = chunk < run_min
        run_col = jnp.where(lt, ccol, run_col)
        run_min = jnp.minimum(chunk, run_min)
    m = jnp.min(run_min, axis=1, keepdims=True)
    cand = jnp.min(jnp.where(run_min == m, run_col, jnp.int32(2**31 - 1)),
                   axis=1, keepdims=True)

    @pl.when(j == 0)
    def _():
        best_ref[...] = m
        bidx_ref[...] = cand

    @pl.when(j > 0)
    def _():
        better = m < best_ref[...]
        bidx_ref[...] = jnp.where(better, cand, bidx_ref[...])
        best_ref[...] = jnp.minimum(m, best_ref[...])

    @pl.when(j == _NJ - 1)
    def _():
        ids_ref[...] = bidx_ref[...].reshape(1, _TM, 1)
        dn_ref[...] = jnp.sqrt(jnp.maximum(best_ref[...], 0.0)).reshape(1, _TM, 1)


def _vq_distance_stage(z2d, codebook):
    return pl.pallas_call(
        _vq_body,
        grid=(_NI, _NJ),
        in_specs=[
            pl.BlockSpec((_TM, _C), lambda i, j: (i, 0)),
            pl.BlockSpec((_K, _C), lambda i, j: (0, 0)),
        ],
        out_specs=[
            pl.BlockSpec((_TM, _TN), lambda i, j: (i, j)),
            pl.BlockSpec((1, _TM, 1), lambda i, j: (i, 0, 0)),
            pl.BlockSpec((1, _TM, 1), lambda i, j: (i, 0, 0)),
        ],
        out_shape=[
            jax.ShapeDtypeStruct((_M, _K), jnp.float32),
            jax.ShapeDtypeStruct((_NI, _TM, 1), jnp.int32),
            jax.ShapeDtypeStruct((_NI, _TM, 1), jnp.float32),
        ],
        scratch_shapes=[
            pltpu.VMEM((_TM, 1), jnp.float32),
            pltpu.VMEM((_TM, 1), jnp.int32),
            pltpu.VMEM((1, _K), jnp.float32),
            pltpu.VMEM((1, _K), jnp.float32),
            pltpu.VMEM((_TM, 1), jnp.float32),
            pltpu.VMEM((_TM, 1), jnp.float32),
        ],
    )(z2d, codebook)


@functools.lru_cache(maxsize=1)
def _build_gather_rows():
    @functools.partial(
        pl.kernel,
        mesh=plsc.VectorSubcoreMesh(core_axis_name="c", subcore_axis_name="s"),
        out_type=jax.ShapeDtypeStruct((_M, _C), jnp.float32),
        scratch_types=[
            pltpu.VMEM((_ROWS_PER_W,), jnp.int32),
            pltpu.VMEM((_ROWS_PER_W, _C), jnp.float32),
            pltpu.SemaphoreType.DMA,
        ],
    )
    def _gather_rows(table_hbm, idx_hbm, out_hbm, idx_v, rows_v, sem):
        wid = lax.axis_index("s") * _NC + lax.axis_index("c")
        base = wid * _ROWS_PER_W
        pltpu.sync_copy(idx_hbm.at[pl.ds(base, _ROWS_PER_W)], idx_v)
        pltpu.async_copy(table_hbm.at[idx_v], rows_v, sem).wait()
        pltpu.sync_copy(rows_v, out_hbm.at[pl.ds(base, _ROWS_PER_W)])

    return _gather_rows


def kernel(z_e, codebook):
    z2d = z_e.reshape(_M, _C)
    sim, ids3, dn3 = _vq_distance_stage(z2d, codebook)
    ids = ids3.reshape(_M)
    z_q = _build_gather_rows()(codebook, ids)
    norms = dn3.reshape(_M)
    loss_vq = (1.0 + _BETA) * jnp.mean(norms)
    return (
        z_q.reshape(_B, _T, _C),
        sim.reshape(_B, _T, _K),
        ids.reshape(_B, _T),
        loss_vq,
    )
